# baseline (speedup 1.0000x reference)
import functools

import jax
import jax.numpy as jnp
from jax import lax
from jax.experimental import pallas as pl
from jax.experimental.pallas import tpu as pltpu
from jax.experimental.pallas import tpu_sc as plsc

B, S, H = 4, 2048, 768
HW = H // 2
EPS = 1e-12
NC, NS = 2, 16
NW = NC * NS
RPW = B * S // NW
GC = 32
NCH = RPW // GC
BR = 1024


def _sc_gather_body(ids_hbm, word_hbm, out_hbm, idx_v, *bufs_and_sems):
    bufs = bufs_and_sems[:NCH]
    gsems = bufs_and_sems[NCH:2 * NCH]
    wsems = bufs_and_sems[2 * NCH:3 * NCH]
    wid = lax.axis_index("s") * NC + lax.axis_index("c")
    base = wid * RPW
    pltpu.sync_copy(ids_hbm.at[pl.ds(base, RPW)], idx_v)

    g = [
        pltpu.async_copy(
            word_hbm.at[idx_v.at[pl.ds(c * GC, GC)]], bufs[c], gsems[c])
        for c in range(NCH)
    ]
    w = []
    for c in range(NCH):
        g[c].wait()
        w.append(pltpu.async_copy(
            bufs[c], out_hbm.at[pl.ds(base + c * GC, GC)], wsems[c]))
    for h in w:
        h.wait()


def _tc_ln_body(g_ref, pos_ref, tok_ref, gamma_ref, beta_ref, o_ref):
    w = g_ref[...]
    lo = lax.bitcast_convert_type(w << 16, jnp.float32)
    hi = lax.bitcast_convert_type(w & jnp.int32(-65536), jnp.float32)
    ylo = lo + pos_ref[:, :HW] + tok_ref[0:1, :HW]
    yhi = hi + pos_ref[:, HW:] + tok_ref[0:1, HW:]
    s1 = jnp.sum(ylo, axis=-1, keepdims=True) \
        + jnp.sum(yhi, axis=-1, keepdims=True)
    mean = s1 * (1.0 / H)
    clo = ylo - mean
    chi = yhi - mean
    s2 = jnp.sum(clo * clo, axis=-1, keepdims=True) \
        + jnp.sum(chi * chi, axis=-1, keepdims=True)
    inv = lax.rsqrt(s2 * (1.0 / H) + EPS)
    o_ref[:, :HW] = clo * inv * gamma_ref[0:1, :HW] + beta_ref[0:1, :HW]
    o_ref[:, HW:] = chi * inv * gamma_ref[0:1, HW:] + beta_ref[0:1, HW:]


@jax.jit
def _embed_ln(ids_flat, word_table, pos_table, tok_table, ln_gamma, ln_beta):
    def pack_bf16_pairs(x):
        xu = lax.bitcast_convert_type(x, jnp.uint32)
        lo, hi = xu[:, :HW], xu[:, HW:]
        rlo = (lo + jnp.uint32(0x7FFF) + ((lo >> 16) & jnp.uint32(1))) >> 16
        rhi = (hi + jnp.uint32(0x7FFF) + ((hi >> 16) & jnp.uint32(1))) \
            & jnp.uint32(0xFFFF0000)
        return lax.bitcast_convert_type(rlo | rhi, jnp.int32)

    word_i32 = pack_bf16_pairs(word_table)

    mesh = plsc.VectorSubcoreMesh(core_axis_name="c", subcore_axis_name="s")
    gathered = pl.kernel(
        _sc_gather_body,
        out_type=jax.ShapeDtypeStruct((B * S, HW), jnp.int32),
        mesh=mesh,
        scratch_types=(
            [pltpu.VMEM((RPW,), jnp.int32)]
            + [pltpu.VMEM((GC, HW), jnp.int32)] * NCH
            + [pltpu.SemaphoreType.DMA] * (2 * NCH)
        ),
    )(ids_flat, word_i32)

    sblk = max(S // BR, 1)
    out = pl.pallas_call(
        _tc_ln_body,
        grid=(sblk, B),
        in_specs=[
            pl.BlockSpec((BR, HW), lambda p, b: (b * sblk + p, 0)),
            pl.BlockSpec((BR, H), lambda p, b: (p, 0)),
            pl.BlockSpec((2, H), lambda p, b: (0, 0)),
            pl.BlockSpec((1, H), lambda p, b: (0, 0)),
            pl.BlockSpec((1, H), lambda p, b: (0, 0)),
        ],
        out_specs=pl.BlockSpec((BR, H), lambda p, b: (b * sblk + p, 0)),
        out_shape=jax.ShapeDtypeStruct((B * S, H), jnp.float32),
    )(gathered, pos_table, tok_table, ln_gamma[None, :], ln_beta[None, :])
    return out


def kernel(input_ids, word_table, pos_table, tok_table, ln_gamma, ln_beta):
    ids_flat = input_ids.reshape(-1)
    out = _embed_ln(ids_flat, word_table, pos_table, tok_table,
                    ln_gamma, ln_beta)
    return out.reshape(B, S, H)


# speedup vs baseline: 1.0440x; 1.0440x over previous
import functools

import jax
import jax.numpy as jnp
from jax import lax
from jax.experimental import pallas as pl
from jax.experimental.pallas import tpu as pltpu
from jax.experimental.pallas import tpu_sc as plsc

B, S, H = 4, 2048, 768
HW = H // 2
EPS = 1e-12
NC, NS = 2, 16
NW = NC * NS
RPW = B * S // NW
GC = 32
NCH = RPW // GC
BR = 2048


def _sc_gather_body(ids_hbm, word_hbm, out_hbm, idx_v, *bufs_and_sems):
    bufs = bufs_and_sems[:NCH]
    gsems = bufs_and_sems[NCH:2 * NCH]
    wsems = bufs_and_sems[2 * NCH:3 * NCH]
    wid = lax.axis_index("s") * NC + lax.axis_index("c")
    base = wid * RPW
    pltpu.sync_copy(ids_hbm.at[pl.ds(base, RPW)], idx_v)

    g = [
        pltpu.async_copy(
            word_hbm.at[idx_v.at[pl.ds(c * GC, GC)]], bufs[c], gsems[c])
        for c in range(NCH)
    ]
    w = []
    for c in range(NCH):
        g[c].wait()
        w.append(pltpu.async_copy(
            bufs[c], out_hbm.at[pl.ds(base + c * GC, GC)], wsems[c]))
    for h in w:
        h.wait()


def _tc_ln_body(g_ref, pos_ref, tok_ref, gamma_ref, beta_ref, o_ref):
    w = g_ref[...]
    lo = lax.bitcast_convert_type(w << 16, jnp.float32)
    hi = lax.bitcast_convert_type(w & jnp.int32(-65536), jnp.float32)
    ylo = lo + pos_ref[:, :HW] + tok_ref[0:1, :HW]
    yhi = hi + pos_ref[:, HW:] + tok_ref[0:1, HW:]
    s1 = jnp.sum(ylo, axis=-1, keepdims=True) \
        + jnp.sum(yhi, axis=-1, keepdims=True)
    mean = s1 * (1.0 / H)
    clo = ylo - mean
    chi = yhi - mean
    s2 = jnp.sum(clo * clo, axis=-1, keepdims=True) \
        + jnp.sum(chi * chi, axis=-1, keepdims=True)
    inv = lax.rsqrt(s2 * (1.0 / H) + EPS)
    o_ref[:, :HW] = clo * inv * gamma_ref[0:1, :HW] + beta_ref[0:1, :HW]
    o_ref[:, HW:] = chi * inv * gamma_ref[0:1, HW:] + beta_ref[0:1, HW:]


@jax.jit
def _embed_ln(ids_flat, word_table, pos_table, tok_table, ln_gamma, ln_beta):
    def pack_bf16_pairs(x):
        xu = lax.bitcast_convert_type(x, jnp.uint32)
        lo, hi = xu[:, :HW], xu[:, HW:]
        rlo = (lo + jnp.uint32(0x7FFF) + ((lo >> 16) & jnp.uint32(1))) >> 16
        rhi = (hi + jnp.uint32(0x7FFF) + ((hi >> 16) & jnp.uint32(1))) \
            & jnp.uint32(0xFFFF0000)
        return lax.bitcast_convert_type(rlo | rhi, jnp.int32)

    word_i32 = pack_bf16_pairs(word_table)

    mesh = plsc.VectorSubcoreMesh(core_axis_name="c", subcore_axis_name="s")
    gathered = pl.kernel(
        _sc_gather_body,
        out_type=jax.ShapeDtypeStruct((B * S, HW), jnp.int32),
        mesh=mesh,
        scratch_types=(
            [pltpu.VMEM((RPW,), jnp.int32)]
            + [pltpu.VMEM((GC, HW), jnp.int32)] * NCH
            + [pltpu.SemaphoreType.DMA] * (2 * NCH)
        ),
    )(ids_flat, word_i32)

    sblk = max(S // BR, 1)
    out = pl.pallas_call(
        _tc_ln_body,
        grid=(sblk, B),
        in_specs=[
            pl.BlockSpec((BR, HW), lambda p, b: (b * sblk + p, 0)),
            pl.BlockSpec((BR, H), lambda p, b: (p, 0)),
            pl.BlockSpec((2, H), lambda p, b: (0, 0)),
            pl.BlockSpec((1, H), lambda p, b: (0, 0)),
            pl.BlockSpec((1, H), lambda p, b: (0, 0)),
        ],
        out_specs=pl.BlockSpec((BR, H), lambda p, b: (b * sblk + p, 0)),
        out_shape=jax.ShapeDtypeStruct((B * S, H), jnp.float32),
    )(gathered, pos_table, tok_table, ln_gamma[None, :], ln_beta[None, :])
    return out


def kernel(input_ids, word_table, pos_table, tok_table, ln_gamma, ln_beta):
    ids_flat = input_ids.reshape(-1)
    out = _embed_ln(ids_flat, word_table, pos_table, tok_table,
                    ln_gamma, ln_beta)
    return out.reshape(B, S, H)
